# Initial kernel scaffold; baseline (speedup 1.0000x reference)
import jax
import jax.numpy as jnp
from jax.experimental import pallas as pl


def kernel(features, adj, weight):
    raise NotImplementedError("write your pallas kernel here")


# speedup vs baseline: 6.1905x; 6.1905x over previous
import functools

import jax
import jax.numpy as jnp
from jax import lax
from jax.experimental import pallas as pl
from jax.experimental.pallas import tpu as pltpu
from jax.experimental.pallas import tpu_sc as plsc

N = 10000
E = 320000
D = 128
NC = 2
NS = 16
NW = NC * NS
EPT = E // NW
CH = 80
NCHUNK = EPT // CH
CTILES = 10
RPT = N // CTILES
ZROWS = 200
LANES = 16


def _sc_agg_body(src_hbm, dst_hbm, feat_hbm, psum_hbm, degp_hbm,
                 idx_s, idx_d, rows_v, zbuf, dzbuf, ones_v, shared_sum,
                 shared_deg, sem):
    c = lax.axis_index("c")
    s = lax.axis_index("s")
    wid = c * NS + s
    zero16 = jnp.zeros((LANES,), jnp.float32)
    ones16 = jnp.ones((LANES,), jnp.float32)

    def zrow(r, carry):
        for j in range(D // LANES):
            zbuf[r, pl.ds(j * LANES, LANES)] = zero16
        return carry
    lax.fori_loop(0, ZROWS, zrow, 0)

    def zdz(i, carry):
        dzbuf[pl.ds(i * LANES, LANES)] = zero16
        return carry
    lax.fori_loop(0, RPT // LANES, zdz, 0)

    for q in range(CH // LANES):
        ones_v[pl.ds(q * LANES, LANES)] = ones16

    row0 = s * RPT

    @pl.when(s < CTILES)
    def _():
        def zshared(k, carry):
            pltpu.sync_copy(zbuf, shared_sum.at[pl.ds(row0 + k * ZROWS, ZROWS)])
            return carry
        lax.fori_loop(0, RPT // ZROWS, zshared, 0)
        pltpu.sync_copy(dzbuf, shared_deg.at[pl.ds(row0, RPT)])

    plsc.subcore_barrier()

    ebase = wid * EPT

    def edge_chunk(j, carry):
        off = ebase + j * CH
        pltpu.sync_copy(src_hbm.at[pl.ds(off, CH)], idx_s)
        pltpu.sync_copy(dst_hbm.at[pl.ds(off, CH)], idx_d)
        pltpu.async_copy(feat_hbm.at[idx_s], rows_v, sem).wait()
        pltpu.sync_copy(rows_v, shared_sum.at[idx_d], add=True)
        pltpu.sync_copy(ones_v, shared_deg.at[idx_d], add=True)
        return carry
    lax.fori_loop(0, NCHUNK, edge_chunk, 0)

    plsc.subcore_barrier()

    @pl.when(s < CTILES)
    def _():
        def cout(k, carry):
            r = row0 + k * ZROWS
            pltpu.sync_copy(shared_sum.at[pl.ds(r, ZROWS)], zbuf)
            pltpu.sync_copy(zbuf, psum_hbm.at[c].at[pl.ds(r, ZROWS)])
            return carry
        lax.fori_loop(0, RPT // ZROWS, cout, 0)
        pltpu.sync_copy(shared_deg.at[pl.ds(row0, RPT)], dzbuf)
        pltpu.sync_copy(dzbuf, degp_hbm.at[pl.ds(c * N + row0, RPT)])


@jax.jit
def _sc_agg(src, dst, features):
    mesh = plsc.VectorSubcoreMesh(core_axis_name="c", subcore_axis_name="s")
    f = pl.kernel(
        _sc_agg_body,
        mesh=mesh,
        out_type=[
            jax.ShapeDtypeStruct((NC, N, D), jnp.float32),
            jax.ShapeDtypeStruct((NC * N,), jnp.float32),
        ],
        scratch_types=[
            pltpu.VMEM((CH,), jnp.int32),
            pltpu.VMEM((CH,), jnp.int32),
            pltpu.VMEM((CH, D), jnp.float32),
            pltpu.VMEM((ZROWS, D), jnp.float32),
            pltpu.VMEM((RPT,), jnp.float32),
            pltpu.VMEM((CH,), jnp.float32),
            pltpu.VMEM_SHARED((N, D), jnp.float32),
            pltpu.VMEM_SHARED((N,), jnp.float32),
            pltpu.SemaphoreType.DMA,
        ],
    )
    return f(src, dst, features)


def _tc_body(f_ref, p_ref, dp_ref, w_ref, o_ref):
    ssum = p_ref[0] + p_ref[1]
    deg = jnp.sum(dp_ref[...], axis=1)
    inv = 1.0 / jnp.maximum(deg, 1.0)
    neigh = ssum * inv[:, None]
    acc = jnp.dot(f_ref[...], w_ref[:D], preferred_element_type=jnp.float32)
    acc += jnp.dot(neigh, w_ref[D:], preferred_element_type=jnp.float32)
    o_ref[...] = jnp.maximum(acc, 0.0)


ROWS_BLK = 1000


@jax.jit
def _tc_combine(features, psum, degp, weight):
    grid = (N // ROWS_BLK,)
    return pl.pallas_call(
        _tc_body,
        grid=grid,
        in_specs=[
            pl.BlockSpec((ROWS_BLK, D), lambda i: (i, 0)),
            pl.BlockSpec((NC, ROWS_BLK, D), lambda i: (0, i, 0)),
            pl.BlockSpec((ROWS_BLK, NC), lambda i: (i, 0)),
            pl.BlockSpec((2 * D, D), lambda i: (0, 0)),
        ],
        out_specs=pl.BlockSpec((ROWS_BLK, D), lambda i: (i, 0)),
        out_shape=jax.ShapeDtypeStruct((N, D), jnp.float32),
    )(features, psum, degp, weight)


def kernel(features, adj, weight):
    adj32 = adj.astype(jnp.int32)
    src = adj32[0]
    dst = adj32[1]
    psum, degp = _sc_agg(src, dst, features)
    degp_t = degp.reshape(NC, N).T
    return _tc_combine(features, psum, degp_t, weight)


# speedup vs baseline: 10.9155x; 1.7633x over previous
import functools

import jax
import jax.numpy as jnp
from jax import lax
from jax.experimental import pallas as pl
from jax.experimental.pallas import tpu as pltpu
from jax.experimental.pallas import tpu_sc as plsc

N = 10000
E = 320000
D = 128
NC = 2
NS = 16
NW = NC * NS
EPT = E // NW
CH = 80
NCHUNK = EPT // CH
CTILES = 10
RPT = N // CTILES
ZROWS = 200
LANES = 16


DEG_LAG = 4


def _sc_agg_body(src_hbm, dst_hbm, feat_hbm, psum_hbm, degp_hbm,
                 sidx_f, didx_f, idx_a, idx_b, idd_v, rows_a, rows_b,
                 dzbuf, ones_v, shared_sum, shared_deg, sem_a, sem_b, dsem):
    c = lax.axis_index("c")
    s = lax.axis_index("s")
    wid = c * NS + s
    zero16 = jnp.zeros((LANES,), jnp.float32)
    ones16 = jnp.ones((LANES,), jnp.float32)

    pltpu.sync_copy(src_hbm.at[pl.ds(wid * EPT, EPT)], sidx_f)
    pltpu.sync_copy(dst_hbm.at[pl.ds(wid * EPT, EPT)], didx_f)

    def zrow(r, carry):
        for j in range(D // LANES):
            rows_a[r, pl.ds(j * LANES, LANES)] = zero16
        return carry
    lax.fori_loop(0, CH, zrow, 0)

    def zdz(i, carry):
        dzbuf[pl.ds(i * LANES, LANES)] = zero16
        return carry
    lax.fori_loop(0, RPT // LANES, zdz, 0)

    for q in range(CH // LANES):
        ones_v[pl.ds(q * LANES, LANES)] = ones16

    row0 = s * RPT

    @pl.when(s < CTILES)
    def _():
        def zshared(k, carry):
            pltpu.sync_copy(rows_a, shared_sum.at[pl.ds(row0 + k * CH, CH)])
            return carry
        lax.fori_loop(0, RPT // CH, zshared, 0)
        rem = RPT - (RPT // CH) * CH
        pltpu.sync_copy(
            rows_a.at[pl.ds(0, rem)],
            shared_sum.at[pl.ds(row0 + (RPT // CH) * CH, rem)])
        pltpu.sync_copy(dzbuf, shared_deg.at[pl.ds(row0, RPT)])

    def cpidx(src_ref, t, ibuf):
        for q in range(CH // LANES):
            ibuf[pl.ds(q * LANES, LANES)] = (
                src_ref[pl.ds(t * CH + q * LANES, LANES)])

    def gstart(t, buf, sem, ibuf):
        cpidx(sidx_f, t, ibuf)
        pltpu.async_copy(feat_hbm.at[ibuf], buf, sem)

    def gwait(t, buf, sem, ibuf):
        pltpu.make_async_copy(feat_hbm.at[ibuf], buf, sem).wait()

    def scat(t, buf, idd):
        cpidx(didx_f, t, idd)
        pltpu.sync_copy(buf, shared_sum.at[idd], add=True)
        pltpu.sync_copy(ones_v, shared_deg.at[idd], add=True)

    gstart(0, rows_a, sem_a, idx_a)
    plsc.subcore_barrier()

    def pair(i, carry):
        t0 = 2 * i
        t1 = t0 + 1
        gwait(t0, rows_a, sem_a, idx_a)
        gstart(t1, rows_b, sem_b, idx_b)
        scat(t0, rows_a, idd_v)
        gwait(t1, rows_b, sem_b, idx_b)
        gstart(t0 + 2, rows_a, sem_a, idx_a)
        scat(t1, rows_b, idd_v)
        return carry
    lax.fori_loop(0, (NCHUNK - 1) // 2, pair, 0)

    t_last = NCHUNK - 1
    gwait(t_last, rows_a, sem_a, idx_a)
    scat(t_last, rows_a, idd_v)

    plsc.subcore_barrier()

    @pl.when(s < CTILES)
    def _():
        def cout(k, carry):
            r = row0 + k * CH
            pltpu.sync_copy(shared_sum.at[pl.ds(r, CH)], rows_a)
            pltpu.sync_copy(rows_a, psum_hbm.at[c].at[pl.ds(r, CH)])
            return carry
        lax.fori_loop(0, RPT // CH, cout, 0)
        rem = RPT - (RPT // CH) * CH
        r_rem = row0 + (RPT // CH) * CH
        pltpu.sync_copy(shared_sum.at[pl.ds(r_rem, rem)],
                        rows_a.at[pl.ds(0, rem)])
        pltpu.sync_copy(rows_a.at[pl.ds(0, rem)],
                        psum_hbm.at[c].at[pl.ds(r_rem, rem)])
        pltpu.sync_copy(shared_deg.at[pl.ds(row0, RPT)], dzbuf)
        pltpu.sync_copy(dzbuf, degp_hbm.at[pl.ds(c * N + row0, RPT)])


@jax.jit
def _sc_agg(src, dst, features):
    mesh = plsc.VectorSubcoreMesh(core_axis_name="c", subcore_axis_name="s")
    f = pl.kernel(
        _sc_agg_body,
        mesh=mesh,
        out_type=[
            jax.ShapeDtypeStruct((NC, N, D), jnp.float32),
            jax.ShapeDtypeStruct((NC * N,), jnp.float32),
        ],
        scratch_types=[
            pltpu.VMEM((EPT,), jnp.int32),
            pltpu.VMEM((EPT,), jnp.int32),
            pltpu.VMEM((CH,), jnp.int32),
            pltpu.VMEM((CH,), jnp.int32),
            pltpu.VMEM((CH,), jnp.int32),
            pltpu.VMEM((CH, D), jnp.float32),
            pltpu.VMEM((CH, D), jnp.float32),
            pltpu.VMEM((RPT,), jnp.float32),
            pltpu.VMEM((CH,), jnp.float32),
            pltpu.VMEM_SHARED((N, D), jnp.float32),
            pltpu.VMEM_SHARED((N,), jnp.float32),
            pltpu.SemaphoreType.DMA,
            pltpu.SemaphoreType.DMA,
            pltpu.SemaphoreType.DMA,
        ],
    )
    return f(src, dst, features)


def _tc_body(f_ref, p_ref, dp_ref, w_ref, o_ref):
    ssum = p_ref[0] + p_ref[1]
    deg = jnp.sum(dp_ref[...], axis=1)
    inv = 1.0 / jnp.maximum(deg, 1.0)
    neigh = ssum * inv[:, None]
    acc = jnp.dot(f_ref[...], w_ref[:D], preferred_element_type=jnp.float32)
    acc += jnp.dot(neigh, w_ref[D:], preferred_element_type=jnp.float32)
    o_ref[...] = jnp.maximum(acc, 0.0)


ROWS_BLK = 1000


@jax.jit
def _tc_combine(features, psum, degp, weight):
    grid = (N // ROWS_BLK,)
    return pl.pallas_call(
        _tc_body,
        grid=grid,
        in_specs=[
            pl.BlockSpec((ROWS_BLK, D), lambda i: (i, 0)),
            pl.BlockSpec((NC, ROWS_BLK, D), lambda i: (0, i, 0)),
            pl.BlockSpec((ROWS_BLK, NC), lambda i: (i, 0)),
            pl.BlockSpec((2 * D, D), lambda i: (0, 0)),
        ],
        out_specs=pl.BlockSpec((ROWS_BLK, D), lambda i: (i, 0)),
        out_shape=jax.ShapeDtypeStruct((N, D), jnp.float32),
    )(features, psum, degp, weight)


def kernel(features, adj, weight):
    adj32 = adj.astype(jnp.int32)
    src = adj32[0]
    dst = adj32[1]
    psum, degp = _sc_agg(src, dst, features)
    degp_t = degp.reshape(NC, N).T
    return _tc_combine(features, psum, degp_t, weight)


# speedup vs baseline: 10.9169x; 1.0001x over previous
import functools

import jax
import jax.numpy as jnp
from jax import lax
from jax.experimental import pallas as pl
from jax.experimental.pallas import tpu as pltpu
from jax.experimental.pallas import tpu_sc as plsc

N = 10000
E = 320000
D = 128
NC = 2
NS = 16
NW = NC * NS
EPT = E // NW
CH = 80
NCHUNK = EPT // CH
CTILES = 10
RPT = N // CTILES
ZROWS = 200
LANES = 16


DEG_LAG = 4


def _sc_agg_body(src_hbm, dst_hbm, feat_hbm, psum_hbm, degp_hbm,
                 sidx_f, didx_f, idx_a, idx_b, idd_v, idd_w, rows_a, rows_b,
                 dzbuf, ones_v, shared_sum, shared_deg, sem_a, sem_b, dsem):
    c = lax.axis_index("c")
    s = lax.axis_index("s")
    wid = c * NS + s
    zero16 = jnp.zeros((LANES,), jnp.float32)
    ones16 = jnp.ones((LANES,), jnp.float32)

    pltpu.sync_copy(src_hbm.at[pl.ds(wid * EPT, EPT)], sidx_f)
    pltpu.sync_copy(dst_hbm.at[pl.ds(wid * EPT, EPT)], didx_f)

    def zrow(r, carry):
        for j in range(D // LANES):
            rows_a[r, pl.ds(j * LANES, LANES)] = zero16
        return carry
    lax.fori_loop(0, CH, zrow, 0)

    def zdz(i, carry):
        dzbuf[pl.ds(i * LANES, LANES)] = zero16
        return carry
    lax.fori_loop(0, RPT // LANES, zdz, 0)

    for q in range(CH // LANES):
        ones_v[pl.ds(q * LANES, LANES)] = ones16

    row0 = s * RPT

    @pl.when(s < CTILES)
    def _():
        def zshared(k, carry):
            pltpu.sync_copy(rows_a, shared_sum.at[pl.ds(row0 + k * CH, CH)])
            return carry
        lax.fori_loop(0, RPT // CH, zshared, 0)
        rem = RPT - (RPT // CH) * CH
        pltpu.sync_copy(
            rows_a.at[pl.ds(0, rem)],
            shared_sum.at[pl.ds(row0 + (RPT // CH) * CH, rem)])
        pltpu.sync_copy(dzbuf, shared_deg.at[pl.ds(row0, RPT)])

    def cpidx(src_ref, t, ibuf):
        for q in range(CH // LANES):
            ibuf[pl.ds(q * LANES, LANES)] = (
                src_ref[pl.ds(t * CH + q * LANES, LANES)])

    def gstart(t, buf, sem, ibuf):
        cpidx(sidx_f, t, ibuf)
        pltpu.async_copy(feat_hbm.at[ibuf], buf, sem)

    def gwait(t, buf, sem, ibuf):
        pltpu.make_async_copy(feat_hbm.at[ibuf], buf, sem).wait()

    def scat(t, buf, idd):
        cpidx(didx_f, t, idd)
        pltpu.sync_copy(buf, shared_sum.at[idd], add=True)
        pltpu.sync_copy(ones_v, shared_deg.at[idd], add=True)

    gstart(0, rows_a, sem_a, idx_a)
    plsc.subcore_barrier()

    def pair(i, carry):
        t0 = 2 * i
        t1 = t0 + 1
        gwait(t0, rows_a, sem_a, idx_a)
        gstart(t1, rows_b, sem_b, idx_b)
        scat(t0, rows_a, idd_v)
        gwait(t1, rows_b, sem_b, idx_b)
        gstart(t0 + 2, rows_a, sem_a, idx_a)
        scat(t1, rows_b, idd_w)
        return carry
    lax.fori_loop(0, (NCHUNK - 1) // 2, pair, 0)

    t_last = NCHUNK - 1
    gwait(t_last, rows_a, sem_a, idx_a)
    scat(t_last, rows_a, idd_v)

    plsc.subcore_barrier()

    @pl.when(s < CTILES)
    def _():
        def cout(k, carry):
            r = row0 + k * CH
            pltpu.sync_copy(shared_sum.at[pl.ds(r, CH)], rows_a)
            pltpu.sync_copy(rows_a, psum_hbm.at[c].at[pl.ds(r, CH)])
            return carry
        lax.fori_loop(0, RPT // CH, cout, 0)
        rem = RPT - (RPT // CH) * CH
        r_rem = row0 + (RPT // CH) * CH
        pltpu.sync_copy(shared_sum.at[pl.ds(r_rem, rem)],
                        rows_a.at[pl.ds(0, rem)])
        pltpu.sync_copy(rows_a.at[pl.ds(0, rem)],
                        psum_hbm.at[c].at[pl.ds(r_rem, rem)])
        pltpu.sync_copy(shared_deg.at[pl.ds(row0, RPT)], dzbuf)
        pltpu.sync_copy(dzbuf, degp_hbm.at[pl.ds(c * N + row0, RPT)])


@jax.jit
def _sc_agg(src, dst, features):
    mesh = plsc.VectorSubcoreMesh(core_axis_name="c", subcore_axis_name="s")
    f = pl.kernel(
        _sc_agg_body,
        mesh=mesh,
        out_type=[
            jax.ShapeDtypeStruct((NC, N, D), jnp.float32),
            jax.ShapeDtypeStruct((NC * N,), jnp.float32),
        ],
        scratch_types=[
            pltpu.VMEM((EPT,), jnp.int32),
            pltpu.VMEM((EPT,), jnp.int32),
            pltpu.VMEM((CH,), jnp.int32),
            pltpu.VMEM((CH,), jnp.int32),
            pltpu.VMEM((CH,), jnp.int32),
            pltpu.VMEM((CH,), jnp.int32),
            pltpu.VMEM((CH, D), jnp.float32),
            pltpu.VMEM((CH, D), jnp.float32),
            pltpu.VMEM((RPT,), jnp.float32),
            pltpu.VMEM((CH,), jnp.float32),
            pltpu.VMEM_SHARED((N, D), jnp.float32),
            pltpu.VMEM_SHARED((N,), jnp.float32),
            pltpu.SemaphoreType.DMA,
            pltpu.SemaphoreType.DMA,
            pltpu.SemaphoreType.DMA,
        ],
    )
    return f(src, dst, features)


def _tc_body(f_ref, p_ref, dp_ref, w_ref, o_ref):
    ssum = p_ref[0] + p_ref[1]
    deg = jnp.sum(dp_ref[...], axis=1)
    inv = 1.0 / jnp.maximum(deg, 1.0)
    neigh = ssum * inv[:, None]
    acc = jnp.dot(f_ref[...], w_ref[:D], preferred_element_type=jnp.float32)
    acc += jnp.dot(neigh, w_ref[D:], preferred_element_type=jnp.float32)
    o_ref[...] = jnp.maximum(acc, 0.0)


ROWS_BLK = 1000


@jax.jit
def _tc_combine(features, psum, degp, weight):
    grid = (N // ROWS_BLK,)
    return pl.pallas_call(
        _tc_body,
        grid=grid,
        in_specs=[
            pl.BlockSpec((ROWS_BLK, D), lambda i: (i, 0)),
            pl.BlockSpec((NC, ROWS_BLK, D), lambda i: (0, i, 0)),
            pl.BlockSpec((ROWS_BLK, NC), lambda i: (i, 0)),
            pl.BlockSpec((2 * D, D), lambda i: (0, 0)),
        ],
        out_specs=pl.BlockSpec((ROWS_BLK, D), lambda i: (i, 0)),
        out_shape=jax.ShapeDtypeStruct((N, D), jnp.float32),
    )(features, psum, degp, weight)


def kernel(features, adj, weight):
    adj32 = adj.astype(jnp.int32)
    src = adj32[0]
    dst = adj32[1]
    psum, degp = _sc_agg(src, dst, features)
    degp_t = degp.reshape(NC, N).T
    return _tc_combine(features, psum, degp_t, weight)
